# baseline (speedup 1.0000x reference)
import functools

import jax
import jax.numpy as jnp
from jax.experimental import pallas as pl
from jax.experimental.pallas import tpu as pltpu

M_BLOCK = 1024
H_BLOCK = 1024


def _gating_kernel(n_h, x_ref, w1_ref, b1_ref, w2_ref, b2_ref, out_ref):
    h_idx = pl.program_id(1)
    h = jax.lax.dot_general(
        x_ref[...], w1_ref[...], (((1,), (0,)), ((), ())),
        preferred_element_type=jnp.float32)
    h = jnp.maximum(h + b1_ref[...], 0.0).astype(jnp.bfloat16)
    part = jax.lax.dot_general(
        h, w2_ref[...], (((1,), (0,)), ((), ())),
        preferred_element_type=jnp.float32)

    @pl.when(h_idx == 0)
    def _init():
        out_ref[...] = part

    @pl.when(h_idx != 0)
    def _acc():
        out_ref[...] += part

    @pl.when(h_idx == n_h - 1)
    def _softmax():
        logits = out_ref[...] + b2_ref[...]
        mx = jnp.max(logits, axis=-1, keepdims=True)
        e = jnp.exp(logits - mx)
        out_ref[...] = e / jnp.sum(e, axis=-1, keepdims=True)


def kernel(inputs, W1, b1, W2, b2):
    M, K = inputs.shape
    H = W1.shape[1]
    E = W2.shape[1]
    n_m = M // M_BLOCK
    n_h = H // H_BLOCK
    return pl.pallas_call(
        functools.partial(_gating_kernel, n_h),
        grid=(n_m, n_h),
        in_specs=[
            pl.BlockSpec((M_BLOCK, K), lambda m, h: (m, 0)),
            pl.BlockSpec((K, H_BLOCK), lambda m, h: (0, h)),
            pl.BlockSpec((1, H_BLOCK), lambda m, h: (0, h)),
            pl.BlockSpec((H_BLOCK, E), lambda m, h: (h, 0)),
            pl.BlockSpec((1, E), lambda m, h: (0, 0)),
        ],
        out_specs=pl.BlockSpec((M_BLOCK, E), lambda m, h: (m, 0)),
        out_shape=jax.ShapeDtypeStruct((M, E), jnp.float32),
        compiler_params=pltpu.CompilerParams(
            dimension_semantics=("parallel", "arbitrary"),
        ),
    )(inputs.astype(jnp.bfloat16), W1.astype(jnp.bfloat16),
      b1.reshape(1, H), W2.astype(jnp.bfloat16), b2.reshape(1, E))


# speedup vs baseline: 1.1714x; 1.1714x over previous
import jax
import jax.numpy as jnp
from jax.experimental import pallas as pl
from jax.experimental.pallas import tpu as pltpu

M_BLOCK = 256


def _gating_kernel(x_ref, w1_ref, b1_ref, w2_ref, b2_ref, out_ref):
    xb = x_ref[...].astype(jnp.bfloat16)
    h = jax.lax.dot_general(
        xb, w1_ref[...], (((1,), (0,)), ((), ())),
        preferred_element_type=jnp.float32)
    h = jnp.maximum(h + b1_ref[...], 0.0).astype(jnp.bfloat16)
    logits = jax.lax.dot_general(
        h, w2_ref[...], (((1,), (0,)), ((), ())),
        preferred_element_type=jnp.float32)
    logits = logits + b2_ref[...]
    mx = jnp.max(logits, axis=-1, keepdims=True)
    e = jnp.exp(logits - mx)
    out_ref[...] = e / jnp.sum(e, axis=-1, keepdims=True)


def kernel(inputs, W1, b1, W2, b2):
    M, K = inputs.shape
    H = W1.shape[1]
    E = W2.shape[1]
    return pl.pallas_call(
        _gating_kernel,
        grid=(M // M_BLOCK,),
        in_specs=[
            pl.BlockSpec((M_BLOCK, K), lambda m: (m, 0)),
            pl.BlockSpec((K, H), lambda m: (0, 0)),
            pl.BlockSpec((1, H), lambda m: (0, 0)),
            pl.BlockSpec((H, E), lambda m: (0, 0)),
            pl.BlockSpec((1, E), lambda m: (0, 0)),
        ],
        out_specs=pl.BlockSpec((M_BLOCK, E), lambda m: (m, 0)),
        out_shape=jax.ShapeDtypeStruct((M, E), jnp.float32),
        compiler_params=pltpu.CompilerParams(
            dimension_semantics=("arbitrary",),
        ),
    )(inputs, W1.astype(jnp.bfloat16), b1.reshape(1, H),
      W2.astype(jnp.bfloat16), b2.reshape(1, E))


# speedup vs baseline: 1.1873x; 1.0136x over previous
import jax
import jax.numpy as jnp
from jax.experimental import pallas as pl
from jax.experimental.pallas import tpu as pltpu

M_BLOCK = 512


def _gating_kernel(x_ref, w1_ref, b1_ref, w2_ref, b2_ref, out_ref):
    xb = x_ref[...].astype(jnp.bfloat16)
    h = jax.lax.dot_general(
        xb, w1_ref[...], (((1,), (0,)), ((), ())),
        preferred_element_type=jnp.float32)
    h = jnp.maximum(h + b1_ref[...], 0.0).astype(jnp.bfloat16)
    logits = jax.lax.dot_general(
        h, w2_ref[...], (((1,), (0,)), ((), ())),
        preferred_element_type=jnp.float32)
    logits = logits + b2_ref[...]
    mx = jnp.max(logits, axis=-1, keepdims=True)
    e = jnp.exp(logits - mx)
    out_ref[...] = e / jnp.sum(e, axis=-1, keepdims=True)


def kernel(inputs, W1, b1, W2, b2):
    M, K = inputs.shape
    H = W1.shape[1]
    E = W2.shape[1]
    return pl.pallas_call(
        _gating_kernel,
        grid=(M // M_BLOCK,),
        in_specs=[
            pl.BlockSpec((M_BLOCK, K), lambda m: (m, 0)),
            pl.BlockSpec((K, H), lambda m: (0, 0)),
            pl.BlockSpec((1, H), lambda m: (0, 0)),
            pl.BlockSpec((H, E), lambda m: (0, 0)),
            pl.BlockSpec((1, E), lambda m: (0, 0)),
        ],
        out_specs=pl.BlockSpec((M_BLOCK, E), lambda m: (m, 0)),
        out_shape=jax.ShapeDtypeStruct((M, E), jnp.float32),
        compiler_params=pltpu.CompilerParams(
            dimension_semantics=("arbitrary",),
        ),
    )(inputs, W1.astype(jnp.bfloat16), b1.reshape(1, H),
      W2.astype(jnp.bfloat16), b2.reshape(1, E))


# speedup vs baseline: 1.2222x; 1.0294x over previous
import functools

import jax
import jax.numpy as jnp
from jax.experimental import pallas as pl
from jax.experimental.pallas import tpu as pltpu

M_BLOCK = 256
W1_CHUNK = 256


def _gating_kernel(np_, x_ref, w1f_ref, b1_ref, w2_ref, b2_ref, out_ref,
                   w1b_ref):
    i = pl.program_id(0)

    @pl.when(i < np_)
    def _cast_w1():
        w1b_ref[:, pl.ds(i * W1_CHUNK, W1_CHUNK)] = (
            w1f_ref[...].astype(jnp.bfloat16))

    @pl.when(i >= np_)
    def _compute():
        xb = x_ref[...].astype(jnp.bfloat16)
        h = jax.lax.dot_general(
            xb, w1b_ref[...], (((1,), (0,)), ((), ())),
            preferred_element_type=jnp.float32)
        h = jnp.maximum(h + b1_ref[...], 0.0).astype(jnp.bfloat16)
        logits = jax.lax.dot_general(
            h, w2_ref[...], (((1,), (0,)), ((), ())),
            preferred_element_type=jnp.float32)
        logits = logits + b2_ref[...]
        mx = jnp.max(logits, axis=-1, keepdims=True)
        e = jnp.exp(logits - mx)
        out_ref[...] = e / jnp.sum(e, axis=-1, keepdims=True)


def kernel(inputs, W1, b1, W2, b2):
    M, K = inputs.shape
    H = W1.shape[1]
    E = W2.shape[1]
    np_ = H // W1_CHUNK
    nm = M // M_BLOCK
    return pl.pallas_call(
        functools.partial(_gating_kernel, np_),
        grid=(np_ + nm,),
        in_specs=[
            pl.BlockSpec((M_BLOCK, K),
                         lambda i: (jnp.maximum(i - np_, 0), 0)),
            pl.BlockSpec((K, W1_CHUNK),
                         lambda i: (0, jnp.minimum(i, np_ - 1))),
            pl.BlockSpec((1, H), lambda i: (0, 0)),
            pl.BlockSpec((H, E), lambda i: (0, 0)),
            pl.BlockSpec((1, E), lambda i: (0, 0)),
        ],
        out_specs=pl.BlockSpec((M_BLOCK, E),
                               lambda i: (jnp.maximum(i - np_, 0), 0)),
        out_shape=jax.ShapeDtypeStruct((M, E), jnp.float32),
        scratch_shapes=[pltpu.VMEM((K, H), jnp.bfloat16)],
        compiler_params=pltpu.CompilerParams(
            dimension_semantics=("arbitrary",),
        ),
    )(inputs, W1, b1.reshape(1, H), W2.astype(jnp.bfloat16),
      b2.reshape(1, E))
